# Initial kernel scaffold; baseline (speedup 1.0000x reference)
import jax
import jax.numpy as jnp
from jax.experimental import pallas as pl


def kernel(x, edge_index, W1, b1, W2, b2, Wd1, bd1, Wd2, bd2, Wo, bo):
    raise NotImplementedError("write your pallas kernel here")


# speedup vs baseline: 7.4164x; 7.4164x over previous
import functools

import jax
import jax.numpy as jnp
from jax import lax
from jax.experimental import pallas as pl
from jax.experimental.pallas import tpu as pltpu
from jax.experimental.pallas import tpu_sc as plsc

N = 100000
E = 1600000
HALF = 50000
TRASH = 50000
ACC_ROWS = 50048
CHUNK = 128
NCH = E // CHUNK
NS = 16
ZCH = 3128

_mesh = plsc.VectorSubcoreMesh(core_axis_name="c", subcore_axis_name="s")
_sc_params = pltpu.CompilerParams(use_tc_tiling_on_sc=False)

f32 = jnp.float32
bf16 = jnp.bfloat16
i32 = jnp.int32


def _zero16():
    return jnp.zeros((16,), f32)


def _n_chunks(s):
    return (NCH - s + NS - 1) // NS


def _local_cols(cidx, cidx2, base):
    for j in range(CHUNK // 16):
        v = cidx[pl.ds(j * 16, 16)]
        inb = (v >= base) & (v < base + HALF)
        cidx2[pl.ds(j * 16, 16)] = jnp.where(inb, v - base, TRASH)


@functools.partial(
    pl.kernel,
    out_type=jax.ShapeDtypeStruct((N,), f32),
    mesh=_mesh,
    compiler_params=_sc_params,
    scratch_types=[
        pltpu.MemorySpace.VMEM_SHARED((ACC_ROWS,), f32),
        pltpu.VMEM((CHUNK,), i32),
        pltpu.VMEM((CHUNK,), i32),
        pltpu.VMEM((CHUNK,), f32),
        pltpu.VMEM((ZCH,), f32),
    ],
)
def _deg(col_hbm, cnt_hbm, acc, cidx, cidx2, ones_v, stage):
    c = lax.axis_index("c")
    s = lax.axis_index("s")
    base = c * HALF

    def zstage(r, _):
        stage[pl.ds(r * 16, 16)] = _zero16()
        return 0

    lax.fori_loop(0, ZCH // 16, zstage, 0)
    for j in range(CHUNK // 16):
        ones_v[pl.ds(j * 16, 16)] = jnp.ones((16,), f32)
    pltpu.sync_copy(stage, acc.at[pl.ds(s * ZCH, ZCH)])
    plsc.subcore_barrier()

    def ebody(i, _):
        off = (s + i * NS) * CHUNK
        pltpu.sync_copy(col_hbm.at[pl.ds(off, CHUNK)], cidx)
        _local_cols(cidx, cidx2, base)
        pltpu.sync_copy(ones_v, acc.at[cidx2], add=True)
        return 0

    lax.fori_loop(0, _n_chunks(s), ebody, 0)
    plsc.subcore_barrier()

    @pl.when(s < NS - 1)
    def _():
        pltpu.sync_copy(acc.at[pl.ds(s * ZCH, ZCH)], stage)
        pltpu.sync_copy(stage, cnt_hbm.at[pl.ds(base + s * ZCH, ZCH)])

    @pl.when(s == NS - 1)
    def _():
        rem = HALF - (NS - 1) * ZCH
        pltpu.sync_copy(acc.at[pl.ds((NS - 1) * ZCH, rem)], stage.at[pl.ds(0, rem)])
        pltpu.sync_copy(stage.at[pl.ds(0, rem)],
                        cnt_hbm.at[pl.ds(base + (NS - 1) * ZCH, rem)])


def _make_agg(npass):
    out_t = [jax.ShapeDtypeStruct((N, 32), bf16)] * npass

    @functools.partial(
        pl.kernel,
        out_type=tuple(out_t) if npass > 1 else out_t[0],
        mesh=_mesh,
        compiler_params=_sc_params,
        scratch_types=[
            pltpu.MemorySpace.VMEM_SHARED((ACC_ROWS, 32), bf16),
            pltpu.VMEM((CHUNK,), i32),
            pltpu.VMEM((CHUNK,), i32),
            pltpu.VMEM((CHUNK,), i32),
            pltpu.VMEM((CHUNK, 32), bf16),
            pltpu.VMEM((ZCH, 32), bf16),
            pltpu.SemaphoreType.DMA,
        ],
    )
    def agg(*refs):
        tables = refs[:npass]
        row_hbm, col_hbm = refs[npass], refs[npass + 1]
        outs = refs[npass + 2:npass + 2 + npass]
        acc, ridx, cidx, cidx2, rows, stage, sem = refs[npass + 2 + npass:]
        c = lax.axis_index("c")
        s = lax.axis_index("s")
        base = c * HALF

        def zstage(r, _):
            stage[r, pl.ds(0, 32)] = jnp.zeros((32,), bf16)
            return 0

        lax.fori_loop(0, ZCH, zstage, 0)

        for p in range(npass):
            if p:
                plsc.subcore_barrier()
            pltpu.sync_copy(stage, acc.at[pl.ds(s * ZCH, ZCH)])
            plsc.subcore_barrier()

            def ebody(i, _):
                off = (s + i * NS) * CHUNK
                pltpu.sync_copy(row_hbm.at[pl.ds(off, CHUNK)], ridx)
                pltpu.sync_copy(col_hbm.at[pl.ds(off, CHUNK)], cidx)
                _local_cols(cidx, cidx2, base)
                pltpu.async_copy(tables[p].at[ridx], rows, sem).wait()
                pltpu.sync_copy(rows, acc.at[cidx2], add=True)
                return 0

            lax.fori_loop(0, _n_chunks(s), ebody, 0)
            plsc.subcore_barrier()

            @pl.when(s < NS - 1)
            def _():
                pltpu.sync_copy(acc.at[pl.ds(s * ZCH, ZCH)], stage)
                pltpu.sync_copy(stage, outs[p].at[pl.ds(base + s * ZCH, ZCH)])

            @pl.when(s == NS - 1)
            def _():
                rem = HALF - (NS - 1) * ZCH
                pltpu.sync_copy(acc.at[pl.ds((NS - 1) * ZCH, rem)],
                                stage.at[pl.ds(0, rem)])
                pltpu.sync_copy(stage.at[pl.ds(0, rem)],
                                outs[p].at[pl.ds(base + (NS - 1) * ZCH, rem)])

    return agg


_agg1 = _make_agg(1)
_agg2 = _make_agg(2)


BM = 5000
NBLK = N // BM


def _tc1_body(x_ref, w_ref, cnt_ref, u1a_ref, u1b_ref, dinv_ref):
    dv = lax.rsqrt(cnt_ref[...] + 1.0)
    u = jnp.dot(x_ref[...], w_ref[...], preferred_element_type=f32) * dv
    u1a_ref[...] = u[:, :32].astype(bf16)
    u1b_ref[...] = u[:, 32:].astype(bf16)
    dinv_ref[...] = dv


def _tc1(x, W1, cnt):
    return pl.pallas_call(
        _tc1_body,
        grid=(NBLK,),
        in_specs=[
            pl.BlockSpec((BM, 28), lambda i: (i, 0)),
            pl.BlockSpec((28, 64), lambda i: (0, 0)),
            pl.BlockSpec((BM, 1), lambda i: (i, 0)),
        ],
        out_specs=[
            pl.BlockSpec((BM, 32), lambda i: (i, 0)),
            pl.BlockSpec((BM, 32), lambda i: (i, 0)),
            pl.BlockSpec((BM, 1), lambda i: (i, 0)),
        ],
        out_shape=[
            jax.ShapeDtypeStruct((N, 32), bf16),
            jax.ShapeDtypeStruct((N, 32), bf16),
            jax.ShapeDtypeStruct((N, 1), f32),
        ],
    )(x, W1, cnt)


def _tc2_body(s1a, s1b, u1a, u1b, dinv, b1, w2, u2_ref):
    dv = dinv[...]
    ha = s1a[...].astype(f32) + u1a[...].astype(f32)
    hb = s1b[...].astype(f32) + u1b[...].astype(f32)
    h = jnp.tanh(jnp.concatenate([ha, hb], axis=1) * dv + b1[...])
    u2_ref[...] = (jnp.dot(h, w2[...], preferred_element_type=f32) * dv).astype(bf16)


def _tc2(s1a, s1b, u1a, u1b, dinv, b1, W2):
    blk32 = pl.BlockSpec((BM, 32), lambda i: (i, 0))
    return pl.pallas_call(
        _tc2_body,
        grid=(NBLK,),
        in_specs=[
            blk32, blk32, blk32, blk32,
            pl.BlockSpec((BM, 1), lambda i: (i, 0)),
            pl.BlockSpec((1, 64), lambda i: (0, 0)),
            pl.BlockSpec((64, 32), lambda i: (0, 0)),
        ],
        out_specs=blk32,
        out_shape=jax.ShapeDtypeStruct((N, 32), bf16),
    )(s1a, s1b, u1a, u1b, dinv, b1, W2)


def _prod_rows(h):
    m = jnp.concatenate([h, jnp.ones((8192 - BM, 32), f32)], axis=0)
    n = 8192
    while n > 1:
        n //= 2
        m = m[:n] * m[n:2 * n]
    return m


def _tc3_body(s2, u2, dinv, b2, wd1, bd1, wd2, bd2, wo, bo,
              out_ref, g_ref, pacc):
    i = pl.program_id(0)
    h = jnp.tanh((s2[...].astype(f32) + u2[...].astype(f32)) * dinv[...]
                 + b2[...])
    part = _prod_rows(h)

    @pl.when(i == 0)
    def _():
        pacc[...] = part

    @pl.when(i > 0)
    def _():
        pacc[...] = pacc[...] * part

    @pl.when(i == NBLK - 1)
    def _():
        g1 = jnp.tanh(jnp.dot(pacc[...], wd1[...], preferred_element_type=f32)
                      + bd1[...])
        g2 = jnp.tanh(jnp.dot(g1, wd2[...], preferred_element_type=f32)
                      + bd2[...])
        out_ref[...] = jnp.dot(g2, wo[...], preferred_element_type=f32) + bo[...]
        g_ref[...] = g2


def _tc3(s2, u2, dinv, b2, Wd1, bd1, Wd2, bd2, Wo, bo):
    blk32 = pl.BlockSpec((BM, 32), lambda i: (i, 0))
    full = lambda a, b: pl.BlockSpec((a, b), lambda i: (0, 0))
    return pl.pallas_call(
        _tc3_body,
        grid=(NBLK,),
        in_specs=[
            blk32, blk32,
            pl.BlockSpec((BM, 1), lambda i: (i, 0)),
            full(1, 32), full(32, 128), full(1, 128),
            full(128, 64), full(1, 64), full(64, 1), full(1, 1),
        ],
        out_specs=[full(1, 1), full(1, 64)],
        out_shape=[
            jax.ShapeDtypeStruct((1, 1), f32),
            jax.ShapeDtypeStruct((1, 64), f32),
        ],
        scratch_shapes=[pltpu.VMEM((1, 32), f32)],
    )(s2, u2, dinv, b2, Wd1, bd1, Wd2, bd2, Wo, bo)


def kernel(x, edge_index, W1, b1, W2, b2, Wd1, bd1, Wd2, bd2, Wo, bo):
    row = edge_index[0]
    col = edge_index[1]
    cnt = _deg(col)
    u1a, u1b, dinv = _tc1(x, W1, cnt.reshape(N, 1))
    s1a, s1b = _agg2(u1a, u1b, row, col)
    u2 = _tc2(s1a, s1b, u1a, u1b, dinv, b1.reshape(1, 64), W2)
    s2 = _agg1(u2, row, col)
    out, g = _tc3(s2, u2, dinv, b2.reshape(1, 32), Wd1, bd1.reshape(1, 128),
                  Wd2, bd2.reshape(1, 64), Wo, bo.reshape(1, 1))
    return (out, g)


# speedup vs baseline: 13.4627x; 1.8153x over previous
import functools

import jax
import jax.numpy as jnp
from jax import lax
from jax.experimental import pallas as pl
from jax.experimental.pallas import tpu as pltpu
from jax.experimental.pallas import tpu_sc as plsc

N = 100000
E = 1600000
HALF = 50000
TRASH = 50000
ACC_ROWS = 50048
CHUNK = 128
NCH = E // CHUNK
NS = 16
ZCH = 3128
NPAIR = 392
CST = 782

_mesh = plsc.VectorSubcoreMesh(core_axis_name="c", subcore_axis_name="s")
_sc_params = pltpu.CompilerParams(use_tc_tiling_on_sc=False)

f32 = jnp.float32
bf16 = jnp.bfloat16
i32 = jnp.int32


def _chunk_bounds(s):
    lo = (s * NCH + NS - 1) // NS
    hi = ((s + 1) * NCH + NS - 1) // NS
    return lo, hi


def _local_cols(cidx, h, cidx2, base, hib):
    for j in range(CHUNK // 16):
        v = cidx[pl.ds(h * CHUNK + j * 16, 16)]
        inb = (v >= base) & (v < hib)
        cidx2[pl.ds(j * 16, 16)] = jnp.where(inb, v - base, TRASH)


@functools.partial(
    pl.kernel,
    out_type=jax.ShapeDtypeStruct((N,), f32),
    mesh=_mesh,
    compiler_params=_sc_params,
    scratch_types=[
        pltpu.MemorySpace.VMEM_SHARED((ACC_ROWS,), f32),
        pltpu.VMEM((CHUNK,), i32),
        pltpu.VMEM((CHUNK,), i32),
        pltpu.VMEM((CHUNK,), f32),
        pltpu.VMEM((ZCH,), f32),
    ],
)
def _deg(col_hbm, cnt_hbm, acc, cidx, cidx2, ones_v, stage):
    c = lax.axis_index("c")
    s = lax.axis_index("s")
    base = c * HALF

    def zstage(r, _):
        stage[pl.ds(r * 16, 16)] = jnp.zeros((16,), f32)
        return 0

    lax.fori_loop(0, ZCH // 16, zstage, 0)
    for j in range(CHUNK // 16):
        ones_v[pl.ds(j * 16, 16)] = jnp.ones((16,), f32)
    pltpu.sync_copy(stage, acc.at[pl.ds(s * ZCH, ZCH)])
    plsc.subcore_barrier()

    lo, hi = _chunk_bounds(s)

    def ebody(i, _):
        off = (lo + i) * CHUNK
        pltpu.sync_copy(col_hbm.at[pl.ds(off, CHUNK)], cidx)
        _local_cols(cidx, 0, cidx2, base, base + HALF)
        pltpu.sync_copy(ones_v, acc.at[cidx2], add=True)
        return 0

    lax.fori_loop(0, hi - lo, ebody, 0)
    plsc.subcore_barrier()

    @pl.when(s < NS - 1)
    def _():
        pltpu.sync_copy(acc.at[pl.ds(s * ZCH, ZCH)], stage)
        pltpu.sync_copy(stage, cnt_hbm.at[pl.ds(base + s * ZCH, ZCH)])

    @pl.when(s == NS - 1)
    def _():
        rem = HALF - (NS - 1) * ZCH
        pltpu.sync_copy(acc.at[pl.ds((NS - 1) * ZCH, rem)], stage.at[pl.ds(0, rem)])
        pltpu.sync_copy(stage.at[pl.ds(0, rem)],
                        cnt_hbm.at[pl.ds(base + (NS - 1) * ZCH, rem)])


def _make_agg(npass):
    out_t = [jax.ShapeDtypeStruct((N, 32), bf16)] * npass

    @functools.partial(
        pl.kernel,
        out_type=tuple(out_t) if npass > 1 else out_t[0],
        mesh=_mesh,
        compiler_params=_sc_params,
        scratch_types=[
            pltpu.MemorySpace.VMEM_SHARED((ACC_ROWS, 32), bf16),
            pltpu.VMEM((2, 2 * CHUNK), i32),
            pltpu.VMEM((2, 2 * CHUNK), i32),
            [pltpu.VMEM((CHUNK,), i32)] * 4,
            [pltpu.VMEM((CHUNK, 32), bf16)] * 4,
            pltpu.VMEM((CST, 32), bf16),
            [pltpu.SemaphoreType.DMA] * 2,
            [pltpu.SemaphoreType.DMA] * 2,
        ],
    )
    def agg(*refs):
        tables = refs[:npass]
        row_hbm, col_hbm = refs[npass], refs[npass + 1]
        outs = refs[npass + 2:npass + 2 + npass]
        (acc, ridx, cidx, cidx2s, rowss, stage, esems,
         gsems) = refs[npass + 2 + npass:]
        c = lax.axis_index("c")
        s = lax.axis_index("s")
        base = c * HALF
        lo, hi = _chunk_bounds(s)
        size = hi - lo

        def eload(p, b):
            cp = jnp.minimum(lo + 2 * p, NCH - 2) * CHUNK
            pltpu.async_copy(row_hbm.at[pl.ds(cp, 2 * CHUNK)], ridx.at[b],
                             esems[b])
            pltpu.async_copy(col_hbm.at[pl.ds(cp, 2 * CHUNK)], cidx.at[b],
                             esems[b])

        def ewait(b):
            pltpu.make_async_copy(row_hbm.at[pl.ds(0, 2 * CHUNK)], ridx.at[b],
                                  esems[b]).wait()
            pltpu.make_async_copy(col_hbm.at[pl.ds(0, 2 * CHUNK)], cidx.at[b],
                                  esems[b]).wait()

        def zstage(r, _):
            stage[r, pl.ds(0, 32)] = jnp.zeros((32,), bf16)
            return 0

        for p in range(npass):
            table, out_hbm = tables[p], outs[p]
            if p:
                plsc.subcore_barrier()
            lax.fori_loop(0, CST, zstage, 0)
            for q in range(4):
                pltpu.sync_copy(stage, acc.at[pl.ds(s * ZCH + q * CST, CST)])
            plsc.subcore_barrier()

            def compute_and_gather(pp, b):
                ewait(b)
                for h in range(2):
                    hib = jnp.where(2 * pp + h < size, base + HALF, base)
                    _local_cols(cidx.at[b], h, cidx2s[2 * b + h], base, hib)
                for h in range(2):
                    pltpu.async_copy(
                        table.at[ridx.at[b, pl.ds(h * CHUNK, CHUNK)]],
                        rowss[2 * b + h], gsems[b])

            def scatter(b):
                for h in range(2):
                    pltpu.make_async_copy(
                        table.at[ridx.at[b, pl.ds(0, CHUNK)]],
                        rowss[2 * b + h], gsems[b]).wait()
                    pltpu.sync_copy(rowss[2 * b + h], acc.at[cidx2s[2 * b + h]],
                                    add=True)

            eload(0, 0)
            compute_and_gather(0, 0)
            eload(1, 1)

            def body(g, _):
                for bb in range(2):
                    pp = 2 * g + 1 + bb
                    b = (1, 0)[bb]
                    compute_and_gather(pp, b)
                    scatter(1 - b)
                    eload(pp + 1, 1 - b)
                return 0

            lax.fori_loop(0, (NPAIR - 2) // 2, body, 0)

            compute_and_gather(NPAIR - 1, 1)
            scatter(0)
            scatter(1)
            plsc.subcore_barrier()

            @pl.when(s < NS - 1)
            def _():
                for q in range(4):
                    r0 = s * ZCH + q * CST
                    pltpu.sync_copy(acc.at[pl.ds(r0, CST)], stage)
                    pltpu.sync_copy(stage, out_hbm.at[pl.ds(base + r0, CST)])

            @pl.when(s == NS - 1)
            def _():
                for q in range(4):
                    r0 = (NS - 1) * ZCH + q * CST
                    sz = CST if q < 3 else HALF - (NS - 1) * ZCH - 3 * CST
                    pltpu.sync_copy(acc.at[pl.ds(r0, sz)],
                                    stage.at[pl.ds(0, sz)])
                    pltpu.sync_copy(stage.at[pl.ds(0, sz)],
                                    out_hbm.at[pl.ds(base + r0, sz)])

    return agg


_agg1 = _make_agg(1)
_agg2 = _make_agg(2)


BM = 5000
NBLK = N // BM


def _tc1_body(x_ref, w_ref, cnt_ref, u1a_ref, u1b_ref, dinv_ref):
    dv = lax.rsqrt(cnt_ref[...] + 1.0)
    u = jnp.dot(x_ref[...], w_ref[...], preferred_element_type=f32) * dv
    u1a_ref[...] = u[:, :32].astype(bf16)
    u1b_ref[...] = u[:, 32:].astype(bf16)
    dinv_ref[...] = dv


def _tc1(x, W1, cnt):
    return pl.pallas_call(
        _tc1_body,
        grid=(NBLK,),
        in_specs=[
            pl.BlockSpec((BM, 28), lambda i: (i, 0)),
            pl.BlockSpec((28, 64), lambda i: (0, 0)),
            pl.BlockSpec((BM, 1), lambda i: (i, 0)),
        ],
        out_specs=[
            pl.BlockSpec((BM, 32), lambda i: (i, 0)),
            pl.BlockSpec((BM, 32), lambda i: (i, 0)),
            pl.BlockSpec((BM, 1), lambda i: (i, 0)),
        ],
        out_shape=[
            jax.ShapeDtypeStruct((N, 32), bf16),
            jax.ShapeDtypeStruct((N, 32), bf16),
            jax.ShapeDtypeStruct((N, 1), f32),
        ],
    )(x, W1, cnt)


def _tc2_body(s1a, s1b, u1a, u1b, dinv, b1, w2, u2_ref):
    dv = dinv[...]
    ha = s1a[...].astype(f32) + u1a[...].astype(f32)
    hb = s1b[...].astype(f32) + u1b[...].astype(f32)
    h = jnp.tanh(jnp.concatenate([ha, hb], axis=1) * dv + b1[...])
    u2_ref[...] = (jnp.dot(h, w2[...], preferred_element_type=f32)
                   * dv).astype(bf16)


def _tc2(s1a, s1b, u1a, u1b, dinv, b1, W2):
    blk32 = pl.BlockSpec((BM, 32), lambda i: (i, 0))
    return pl.pallas_call(
        _tc2_body,
        grid=(NBLK,),
        in_specs=[
            blk32, blk32, blk32, blk32,
            pl.BlockSpec((BM, 1), lambda i: (i, 0)),
            pl.BlockSpec((1, 64), lambda i: (0, 0)),
            pl.BlockSpec((64, 32), lambda i: (0, 0)),
        ],
        out_specs=blk32,
        out_shape=jax.ShapeDtypeStruct((N, 32), bf16),
    )(s1a, s1b, u1a, u1b, dinv, b1, W2)


def _prod_rows(h):
    m = jnp.concatenate([h, jnp.ones((8192 - BM, 32), f32)], axis=0)
    n = 8192
    while n > 1:
        n //= 2
        m = m[:n] * m[n:2 * n]
    return m


def _tc3_body(s2, u2, dinv, b2, wd1, bd1, wd2, bd2, wo, bo,
              out_ref, g_ref, pacc):
    i = pl.program_id(0)
    h = jnp.tanh((s2[...].astype(f32) + u2[...].astype(f32)) * dinv[...]
                 + b2[...])
    part = _prod_rows(h)

    @pl.when(i == 0)
    def _():
        pacc[...] = part

    @pl.when(i > 0)
    def _():
        pacc[...] = pacc[...] * part

    @pl.when(i == NBLK - 1)
    def _():
        g1 = jnp.tanh(jnp.dot(pacc[...], wd1[...], preferred_element_type=f32)
                      + bd1[...])
        g2 = jnp.tanh(jnp.dot(g1, wd2[...], preferred_element_type=f32)
                      + bd2[...])
        out_ref[...] = jnp.dot(g2, wo[...], preferred_element_type=f32) + bo[...]
        g_ref[...] = g2


def _tc3(s2, u2, dinv, b2, Wd1, bd1, Wd2, bd2, Wo, bo):
    blk32 = pl.BlockSpec((BM, 32), lambda i: (i, 0))
    full = lambda a, b: pl.BlockSpec((a, b), lambda i: (0, 0))
    return pl.pallas_call(
        _tc3_body,
        grid=(NBLK,),
        in_specs=[
            blk32, blk32,
            pl.BlockSpec((BM, 1), lambda i: (i, 0)),
            full(1, 32), full(32, 128), full(1, 128),
            full(128, 64), full(1, 64), full(64, 1), full(1, 1),
        ],
        out_specs=[full(1, 1), full(1, 64)],
        out_shape=[
            jax.ShapeDtypeStruct((1, 1), f32),
            jax.ShapeDtypeStruct((1, 64), f32),
        ],
        scratch_shapes=[pltpu.VMEM((1, 32), f32)],
    )(s2, u2, dinv, b2, Wd1, bd1, Wd2, bd2, Wo, bo)


def kernel(x, edge_index, W1, b1, W2, b2, Wd1, bd1, Wd2, bd2, Wo, bo):
    row = edge_index[0]
    col = edge_index[1]
    cnt = _deg(col)
    u1a, u1b, dinv = _tc1(x, W1, cnt.reshape(N, 1))
    s1a, s1b = _agg2(u1a, u1b, row, col)
    u2 = _tc2(s1a, s1b, u1a, u1b, dinv, b1.reshape(1, 64), W2)
    s2 = _agg1(u2, row, col)
    out, g = _tc3(s2, u2, dinv, b2.reshape(1, 32), Wd1, bd1.reshape(1, 128),
                  Wd2, bd2.reshape(1, 64), Wo, bo.reshape(1, 1))
    return (out, g)


# speedup vs baseline: 13.5499x; 1.0065x over previous
import functools

import jax
import jax.numpy as jnp
from jax import lax
from jax.experimental import pallas as pl
from jax.experimental.pallas import tpu as pltpu
from jax.experimental.pallas import tpu_sc as plsc

N = 100000
E = 1600000
HALF = 50000
TRASH = 50000
ACC_ROWS = 50048
CHUNK = 128
NCH = E // CHUNK
NS = 16
ZCH = 3128
NPAIR = 392
CST = 782

_mesh = plsc.VectorSubcoreMesh(core_axis_name="c", subcore_axis_name="s")
_sc_params = pltpu.CompilerParams(use_tc_tiling_on_sc=False)

f32 = jnp.float32
bf16 = jnp.bfloat16
i32 = jnp.int32


def _chunk_bounds(s):
    lo = (s * NCH + NS - 1) // NS
    hi = ((s + 1) * NCH + NS - 1) // NS
    return lo, hi


def _local_cols(cidx, h, cidx2, base, hib):
    for j in range(CHUNK // 16):
        v = cidx[pl.ds(h * CHUNK + j * 16, 16)]
        inb = (v >= base) & (v < hib)
        cidx2[pl.ds(j * 16, 16)] = jnp.where(inb, v - base, TRASH)


@functools.partial(
    pl.kernel,
    out_type=jax.ShapeDtypeStruct((N,), f32),
    mesh=_mesh,
    compiler_params=_sc_params,
    scratch_types=[
        pltpu.MemorySpace.VMEM_SHARED((ACC_ROWS,), f32),
        pltpu.VMEM((2, 2 * CHUNK), i32),
        [pltpu.VMEM((CHUNK,), i32)] * 4,
        pltpu.VMEM((CHUNK,), f32),
        pltpu.VMEM((ZCH,), f32),
        [pltpu.SemaphoreType.DMA] * 2,
        [pltpu.SemaphoreType.DMA] * 2,
    ],
)
def _deg(col_hbm, cnt_hbm, acc, cidx, cidx2s, ones_v, stage, esems, ssems):
    c = lax.axis_index("c")
    s = lax.axis_index("s")
    base = c * HALF

    def zstage(r, _):
        stage[pl.ds(r * 16, 16)] = jnp.zeros((16,), f32)
        return 0

    lax.fori_loop(0, ZCH // 16, zstage, 0)
    for j in range(CHUNK // 16):
        ones_v[pl.ds(j * 16, 16)] = jnp.ones((16,), f32)
    pltpu.sync_copy(stage, acc.at[pl.ds(s * ZCH, ZCH)])
    plsc.subcore_barrier()

    lo, hi = _chunk_bounds(s)
    size = hi - lo

    def eload(p, b):
        cp = jnp.minimum(lo + 2 * p, NCH - 2) * CHUNK
        pltpu.async_copy(col_hbm.at[pl.ds(cp, 2 * CHUNK)], cidx.at[b],
                         esems[b])

    def ewait(b):
        pltpu.make_async_copy(col_hbm.at[pl.ds(0, 2 * CHUNK)], cidx.at[b],
                              esems[b]).wait()

    def compute(p, b):
        for h in range(2):
            hib = jnp.where(2 * p + h < size, base + HALF, base)
            _local_cols(cidx.at[b], h, cidx2s[2 * b + h], base, hib)

    def fire_scatters(b):
        for h in range(2):
            pltpu.async_copy(ones_v, acc.at[cidx2s[2 * b + h]], ssems[b],
                             add=True)

    def swait(b):
        for h in range(2):
            pltpu.make_async_copy(ones_v, acc.at[cidx2s[2 * b + h]],
                                  ssems[b]).wait()

    eload(0, 0)
    eload(1, 1)
    ewait(0)
    compute(0, 0)
    fire_scatters(0)
    ewait(1)
    compute(1, 1)
    fire_scatters(1)
    eload(2, 0)

    def body(g, _):
        for b in range(2):
            p = 2 * g + 2 + b
            ewait(b)
            swait(b)
            compute(p, b)
            fire_scatters(b)
            eload(p + 1, 1 - b)
        return 0

    lax.fori_loop(0, (NPAIR - 2) // 2, body, 0)
    swait(0)
    swait(1)
    ewait(0)
    plsc.subcore_barrier()

    @pl.when(s < NS - 1)
    def _():
        pltpu.sync_copy(acc.at[pl.ds(s * ZCH, ZCH)], stage)
        pltpu.sync_copy(stage, cnt_hbm.at[pl.ds(base + s * ZCH, ZCH)])

    @pl.when(s == NS - 1)
    def _():
        rem = HALF - (NS - 1) * ZCH
        pltpu.sync_copy(acc.at[pl.ds((NS - 1) * ZCH, rem)], stage.at[pl.ds(0, rem)])
        pltpu.sync_copy(stage.at[pl.ds(0, rem)],
                        cnt_hbm.at[pl.ds(base + (NS - 1) * ZCH, rem)])


def _make_agg(npass):
    out_t = [jax.ShapeDtypeStruct((N, 32), bf16)] * npass

    @functools.partial(
        pl.kernel,
        out_type=tuple(out_t) if npass > 1 else out_t[0],
        mesh=_mesh,
        compiler_params=_sc_params,
        scratch_types=[
            pltpu.MemorySpace.VMEM_SHARED((ACC_ROWS, 32), bf16),
            pltpu.VMEM((2, 2 * CHUNK), i32),
            pltpu.VMEM((2, 2 * CHUNK), i32),
            [pltpu.VMEM((CHUNK,), i32)] * 4,
            [pltpu.VMEM((CHUNK, 32), bf16)] * 4,
            pltpu.VMEM((CST, 32), bf16),
            [pltpu.SemaphoreType.DMA] * 2,
            [pltpu.SemaphoreType.DMA] * 2,
            [pltpu.SemaphoreType.DMA] * 2,
        ],
    )
    def agg(*refs):
        tables = refs[:npass]
        row_hbm, col_hbm = refs[npass], refs[npass + 1]
        outs = refs[npass + 2:npass + 2 + npass]
        (acc, ridx, cidx, cidx2s, rowss, stage, esems, gsems,
         ssems) = refs[npass + 2 + npass:]
        c = lax.axis_index("c")
        s = lax.axis_index("s")
        base = c * HALF
        lo, hi = _chunk_bounds(s)
        size = hi - lo

        def eload(p, b):
            cp = jnp.minimum(lo + 2 * p, NCH - 2) * CHUNK
            pltpu.async_copy(row_hbm.at[pl.ds(cp, 2 * CHUNK)], ridx.at[b],
                             esems[b])
            pltpu.async_copy(col_hbm.at[pl.ds(cp, 2 * CHUNK)], cidx.at[b],
                             esems[b])

        def ewait(b):
            pltpu.make_async_copy(row_hbm.at[pl.ds(0, 2 * CHUNK)], ridx.at[b],
                                  esems[b]).wait()
            pltpu.make_async_copy(col_hbm.at[pl.ds(0, 2 * CHUNK)], cidx.at[b],
                                  esems[b]).wait()

        def zstage(r, _):
            stage[r, pl.ds(0, 32)] = jnp.zeros((32,), bf16)
            return 0

        for p in range(npass):
            table, out_hbm = tables[p], outs[p]
            if p:
                plsc.subcore_barrier()
            lax.fori_loop(0, CST, zstage, 0)
            for q in range(4):
                pltpu.sync_copy(stage, acc.at[pl.ds(s * ZCH + q * CST, CST)])
            plsc.subcore_barrier()

            def compute_and_gather(pp, b):
                ewait(b)
                for h in range(2):
                    hib = jnp.where(2 * pp + h < size, base + HALF, base)
                    _local_cols(cidx.at[b], h, cidx2s[2 * b + h], base, hib)
                for h in range(2):
                    pltpu.async_copy(
                        table.at[ridx.at[b, pl.ds(h * CHUNK, CHUNK)]],
                        rowss[2 * b + h], gsems[b])

            def gwait(b):
                for h in range(2):
                    pltpu.make_async_copy(
                        table.at[ridx.at[b, pl.ds(0, CHUNK)]],
                        rowss[2 * b + h], gsems[b]).wait()

            def fire_scatters(b):
                for h in range(2):
                    pltpu.async_copy(rowss[2 * b + h],
                                     acc.at[cidx2s[2 * b + h]], ssems[b],
                                     add=True)

            def swait(b):
                for h in range(2):
                    pltpu.make_async_copy(rowss[2 * b + h],
                                          acc.at[cidx2s[2 * b + h]],
                                          ssems[b]).wait()

            eload(0, 0)
            eload(1, 1)
            compute_and_gather(0, 0)
            compute_and_gather(1, 1)
            gwait(0)
            fire_scatters(0)
            eload(2, 0)

            def body(g, _):
                for b in range(2):
                    pp = 2 * g + 2 + b
                    swait(b)
                    compute_and_gather(pp, b)
                    gwait(1 - b)
                    fire_scatters(1 - b)
                    eload(pp + 1, 1 - b)
                return 0

            lax.fori_loop(0, (NPAIR - 2) // 2, body, 0)

            gwait(1)
            fire_scatters(1)
            swait(0)
            swait(1)
            ewait(0)
            plsc.subcore_barrier()

            @pl.when(s < NS - 1)
            def _():
                for q in range(4):
                    r0 = s * ZCH + q * CST
                    pltpu.sync_copy(acc.at[pl.ds(r0, CST)], stage)
                    pltpu.sync_copy(stage, out_hbm.at[pl.ds(base + r0, CST)])

            @pl.when(s == NS - 1)
            def _():
                for q in range(4):
                    r0 = (NS - 1) * ZCH + q * CST
                    sz = CST if q < 3 else HALF - (NS - 1) * ZCH - 3 * CST
                    pltpu.sync_copy(acc.at[pl.ds(r0, sz)],
                                    stage.at[pl.ds(0, sz)])
                    pltpu.sync_copy(stage.at[pl.ds(0, sz)],
                                    out_hbm.at[pl.ds(base + r0, sz)])

    return agg


_agg1 = _make_agg(1)
_agg2 = _make_agg(2)


BM = 5000
NBLK = N // BM


def _tc1_body(x_ref, w_ref, cnt_ref, u1a_ref, u1b_ref, dinv_ref):
    dv = lax.rsqrt(cnt_ref[...] + 1.0)
    u = jnp.dot(x_ref[...], w_ref[...], preferred_element_type=f32) * dv
    u1a_ref[...] = u[:, :32].astype(bf16)
    u1b_ref[...] = u[:, 32:].astype(bf16)
    dinv_ref[...] = dv


def _tc1(x, W1, cnt):
    return pl.pallas_call(
        _tc1_body,
        grid=(NBLK,),
        in_specs=[
            pl.BlockSpec((BM, 28), lambda i: (i, 0)),
            pl.BlockSpec((28, 64), lambda i: (0, 0)),
            pl.BlockSpec((BM, 1), lambda i: (i, 0)),
        ],
        out_specs=[
            pl.BlockSpec((BM, 32), lambda i: (i, 0)),
            pl.BlockSpec((BM, 32), lambda i: (i, 0)),
            pl.BlockSpec((BM, 1), lambda i: (i, 0)),
        ],
        out_shape=[
            jax.ShapeDtypeStruct((N, 32), bf16),
            jax.ShapeDtypeStruct((N, 32), bf16),
            jax.ShapeDtypeStruct((N, 1), f32),
        ],
    )(x, W1, cnt)


def _tc2_body(s1a, s1b, u1a, u1b, dinv, b1, w2, u2_ref):
    dv = dinv[...]
    ha = s1a[...].astype(f32) + u1a[...].astype(f32)
    hb = s1b[...].astype(f32) + u1b[...].astype(f32)
    h = jnp.tanh(jnp.concatenate([ha, hb], axis=1) * dv + b1[...])
    u2_ref[...] = (jnp.dot(h, w2[...], preferred_element_type=f32)
                   * dv).astype(bf16)


def _tc2(s1a, s1b, u1a, u1b, dinv, b1, W2):
    blk32 = pl.BlockSpec((BM, 32), lambda i: (i, 0))
    return pl.pallas_call(
        _tc2_body,
        grid=(NBLK,),
        in_specs=[
            blk32, blk32, blk32, blk32,
            pl.BlockSpec((BM, 1), lambda i: (i, 0)),
            pl.BlockSpec((1, 64), lambda i: (0, 0)),
            pl.BlockSpec((64, 32), lambda i: (0, 0)),
        ],
        out_specs=blk32,
        out_shape=jax.ShapeDtypeStruct((N, 32), bf16),
    )(s1a, s1b, u1a, u1b, dinv, b1, W2)


def _prod_rows(h):
    m = jnp.concatenate([h, jnp.ones((8192 - BM, 32), f32)], axis=0)
    n = 8192
    while n > 1:
        n //= 2
        m = m[:n] * m[n:2 * n]
    return m


def _tc3_body(s2, u2, dinv, b2, wd1, bd1, wd2, bd2, wo, bo,
              out_ref, g_ref, pacc):
    i = pl.program_id(0)
    h = jnp.tanh((s2[...].astype(f32) + u2[...].astype(f32)) * dinv[...]
                 + b2[...])
    part = _prod_rows(h)

    @pl.when(i == 0)
    def _():
        pacc[...] = part

    @pl.when(i > 0)
    def _():
        pacc[...] = pacc[...] * part

    @pl.when(i == NBLK - 1)
    def _():
        g1 = jnp.tanh(jnp.dot(pacc[...], wd1[...], preferred_element_type=f32)
                      + bd1[...])
        g2 = jnp.tanh(jnp.dot(g1, wd2[...], preferred_element_type=f32)
                      + bd2[...])
        out_ref[...] = jnp.dot(g2, wo[...], preferred_element_type=f32) + bo[...]
        g_ref[...] = g2


def _tc3(s2, u2, dinv, b2, Wd1, bd1, Wd2, bd2, Wo, bo):
    blk32 = pl.BlockSpec((BM, 32), lambda i: (i, 0))
    full = lambda a, b: pl.BlockSpec((a, b), lambda i: (0, 0))
    return pl.pallas_call(
        _tc3_body,
        grid=(NBLK,),
        in_specs=[
            blk32, blk32,
            pl.BlockSpec((BM, 1), lambda i: (i, 0)),
            full(1, 32), full(32, 128), full(1, 128),
            full(128, 64), full(1, 64), full(64, 1), full(1, 1),
        ],
        out_specs=[full(1, 1), full(1, 64)],
        out_shape=[
            jax.ShapeDtypeStruct((1, 1), f32),
            jax.ShapeDtypeStruct((1, 64), f32),
        ],
        scratch_shapes=[pltpu.VMEM((1, 32), f32)],
    )(s2, u2, dinv, b2, Wd1, bd1, Wd2, bd2, Wo, bo)


def kernel(x, edge_index, W1, b1, W2, b2, Wd1, bd1, Wd2, bd2, Wo, bo):
    row = edge_index[0]
    col = edge_index[1]
    cnt = _deg(col)
    u1a, u1b, dinv = _tc1(x, W1, cnt.reshape(N, 1))
    s1a, s1b = _agg2(u1a, u1b, row, col)
    u2 = _tc2(s1a, s1b, u1a, u1b, dinv, b1.reshape(1, 64), W2)
    s2 = _agg1(u2, row, col)
    out, g = _tc3(s2, u2, dinv, b2.reshape(1, 32), Wd1, bd1.reshape(1, 128),
                  Wd2, bd2.reshape(1, 64), Wo, bo.reshape(1, 1))
    return (out, g)


# speedup vs baseline: 17.9006x; 1.3211x over previous
import functools

import jax
import jax.numpy as jnp
from jax import lax
from jax.experimental import pallas as pl
from jax.experimental.pallas import tpu as pltpu
from jax.experimental.pallas import tpu_sc as plsc

N = 100000
E = 1600000
HALF = 50000
TRASH = 50000
ACC_ROWS = 50048
CHUNK = 128
NCH = E // CHUNK
NS = 16
ZCH = 3128
NPAIR = 392
CST = 782

_mesh = plsc.VectorSubcoreMesh(core_axis_name="c", subcore_axis_name="s")
_sc_params = pltpu.CompilerParams(use_tc_tiling_on_sc=False)

f32 = jnp.float32
bf16 = jnp.bfloat16
i32 = jnp.int32


def _chunk_bounds(s):
    lo = (s * NCH + NS - 1) // NS
    hi = ((s + 1) * NCH + NS - 1) // NS
    return lo, hi


def _local_cols(cidx, h, cidx2, base, hib, trash=TRASH):
    for j in range(CHUNK // 16):
        v = cidx[pl.ds(h * CHUNK + j * 16, 16)]
        inb = (v >= base) & (v < hib)
        cidx2[pl.ds(j * 16, 16)] = jnp.where(inb, v - base, trash)


DNCH = NCH // 2
DACC = 100096
DZCH = 6256
DTRASH = N
DNPAIR = 196


@functools.partial(
    pl.kernel,
    out_type=jax.ShapeDtypeStruct((2, N), f32),
    mesh=_mesh,
    compiler_params=_sc_params,
    scratch_types=[
        pltpu.MemorySpace.VMEM_SHARED((DACC,), f32),
        pltpu.VMEM((2, 2 * CHUNK), i32),
        [pltpu.VMEM((CHUNK,), i32)] * 4,
        pltpu.VMEM((CHUNK,), f32),
        pltpu.VMEM((DZCH,), f32),
        [pltpu.SemaphoreType.DMA] * 2,
        [pltpu.SemaphoreType.DMA] * 2,
    ],
)
def _deg(col_hbm, cnt_hbm, acc, cidx, cidx2s, ones_v, stage, esems, ssems):
    c = lax.axis_index("c")
    s = lax.axis_index("s")

    def zstage(r, _):
        stage[pl.ds(r * 16, 16)] = jnp.zeros((16,), f32)
        return 0

    lax.fori_loop(0, DZCH // 16, zstage, 0)
    for j in range(CHUNK // 16):
        ones_v[pl.ds(j * 16, 16)] = jnp.ones((16,), f32)
    pltpu.sync_copy(stage, acc.at[pl.ds(s * DZCH, DZCH)])
    plsc.subcore_barrier()

    lo = c * DNCH + (s * DNCH + NS - 1) // NS
    hi = c * DNCH + ((s + 1) * DNCH + NS - 1) // NS
    size = hi - lo

    def eload(p, b):
        cp = jnp.minimum(lo + 2 * p, (c + 1) * DNCH - 2) * CHUNK
        pltpu.async_copy(col_hbm.at[pl.ds(cp, 2 * CHUNK)], cidx.at[b],
                         esems[b])

    def ewait(b):
        pltpu.make_async_copy(col_hbm.at[pl.ds(0, 2 * CHUNK)], cidx.at[b],
                              esems[b]).wait()

    def compute(p, b):
        for h in range(2):
            hib = jnp.where(2 * p + h < size, N, 0)
            _local_cols(cidx.at[b], h, cidx2s[2 * b + h], 0, hib, DTRASH)

    def fire_scatters(b):
        for h in range(2):
            pltpu.async_copy(ones_v, acc.at[cidx2s[2 * b + h]], ssems[b],
                             add=True)

    def swait(b):
        for h in range(2):
            pltpu.make_async_copy(ones_v, acc.at[cidx2s[2 * b + h]],
                                  ssems[b]).wait()

    eload(0, 0)
    eload(1, 1)
    ewait(0)
    compute(0, 0)
    fire_scatters(0)
    ewait(1)
    compute(1, 1)
    fire_scatters(1)
    eload(2, 0)

    def body(g, _):
        for b in range(2):
            p = 2 * g + 2 + b
            ewait(b)
            swait(b)
            compute(p, b)
            fire_scatters(b)
            eload(p + 1, 1 - b)
        return 0

    lax.fori_loop(0, (DNPAIR - 2) // 2, body, 0)
    swait(0)
    swait(1)
    ewait(0)
    plsc.subcore_barrier()

    @pl.when(s < NS - 1)
    def _():
        pltpu.sync_copy(acc.at[pl.ds(s * DZCH, DZCH)], stage)
        pltpu.sync_copy(stage, cnt_hbm.at[c, pl.ds(s * DZCH, DZCH)])

    @pl.when(s == NS - 1)
    def _():
        rem = N - (NS - 1) * DZCH
        pltpu.sync_copy(acc.at[pl.ds((NS - 1) * DZCH, rem)], stage.at[pl.ds(0, rem)])
        pltpu.sync_copy(stage.at[pl.ds(0, rem)],
                        cnt_hbm.at[c, pl.ds((NS - 1) * DZCH, rem)])


def _make_agg(npass):
    out_t = [jax.ShapeDtypeStruct((N, 32), bf16)] * npass

    @functools.partial(
        pl.kernel,
        out_type=tuple(out_t) if npass > 1 else out_t[0],
        mesh=_mesh,
        compiler_params=_sc_params,
        scratch_types=[
            pltpu.MemorySpace.VMEM_SHARED((ACC_ROWS, 32), bf16),
            pltpu.VMEM((2, 2 * CHUNK), i32),
            pltpu.VMEM((2, 2 * CHUNK), i32),
            [pltpu.VMEM((CHUNK,), i32)] * 4,
            [pltpu.VMEM((CHUNK, 32), bf16)] * 4,
            pltpu.VMEM((CST, 32), bf16),
            [pltpu.SemaphoreType.DMA] * 2,
            [pltpu.SemaphoreType.DMA] * 2,
            [pltpu.SemaphoreType.DMA] * 2,
        ],
    )
    def agg(*refs):
        tables = refs[:npass]
        row_hbm, col_hbm = refs[npass], refs[npass + 1]
        outs = refs[npass + 2:npass + 2 + npass]
        (acc, ridx, cidx, cidx2s, rowss, stage, esems, gsems,
         ssems) = refs[npass + 2 + npass:]
        c = lax.axis_index("c")
        s = lax.axis_index("s")
        base = c * HALF
        lo, hi = _chunk_bounds(s)
        size = hi - lo

        def eload(p, b):
            cp = jnp.minimum(lo + 2 * p, NCH - 2) * CHUNK
            pltpu.async_copy(row_hbm.at[pl.ds(cp, 2 * CHUNK)], ridx.at[b],
                             esems[b])
            pltpu.async_copy(col_hbm.at[pl.ds(cp, 2 * CHUNK)], cidx.at[b],
                             esems[b])

        def ewait(b):
            pltpu.make_async_copy(row_hbm.at[pl.ds(0, 2 * CHUNK)], ridx.at[b],
                                  esems[b]).wait()
            pltpu.make_async_copy(col_hbm.at[pl.ds(0, 2 * CHUNK)], cidx.at[b],
                                  esems[b]).wait()

        def zstage(r, _):
            stage[r, pl.ds(0, 32)] = jnp.zeros((32,), bf16)
            return 0

        for p in range(npass):
            table, out_hbm = tables[p], outs[p]
            if p:
                plsc.subcore_barrier()
            lax.fori_loop(0, CST, zstage, 0)
            for q in range(4):
                pltpu.sync_copy(stage, acc.at[pl.ds(s * ZCH + q * CST, CST)])
            plsc.subcore_barrier()

            def compute_and_gather(pp, b):
                ewait(b)
                for h in range(2):
                    hib = jnp.where(2 * pp + h < size, base + HALF, base)
                    _local_cols(cidx.at[b], h, cidx2s[2 * b + h], base, hib)
                for h in range(2):
                    pltpu.async_copy(
                        table.at[ridx.at[b, pl.ds(h * CHUNK, CHUNK)]],
                        rowss[2 * b + h], gsems[b])

            def gwait(b):
                for h in range(2):
                    pltpu.make_async_copy(
                        table.at[ridx.at[b, pl.ds(0, CHUNK)]],
                        rowss[2 * b + h], gsems[b]).wait()

            def fire_scatters(b):
                for h in range(2):
                    pltpu.async_copy(rowss[2 * b + h],
                                     acc.at[cidx2s[2 * b + h]], ssems[b],
                                     add=True)

            def swait(b):
                for h in range(2):
                    pltpu.make_async_copy(rowss[2 * b + h],
                                          acc.at[cidx2s[2 * b + h]],
                                          ssems[b]).wait()

            eload(0, 0)
            eload(1, 1)
            compute_and_gather(0, 0)
            compute_and_gather(1, 1)
            gwait(0)
            fire_scatters(0)
            eload(2, 0)

            def body(g, _):
                for b in range(2):
                    pp = 2 * g + 2 + b
                    swait(b)
                    compute_and_gather(pp, b)
                    gwait(1 - b)
                    fire_scatters(1 - b)
                    eload(pp + 1, 1 - b)
                return 0

            lax.fori_loop(0, (NPAIR - 2) // 2, body, 0)

            gwait(1)
            fire_scatters(1)
            swait(0)
            swait(1)
            ewait(0)
            plsc.subcore_barrier()

            @pl.when(s < NS - 1)
            def _():
                for q in range(4):
                    r0 = s * ZCH + q * CST
                    pltpu.sync_copy(acc.at[pl.ds(r0, CST)], stage)
                    pltpu.sync_copy(stage, out_hbm.at[pl.ds(base + r0, CST)])

            @pl.when(s == NS - 1)
            def _():
                for q in range(4):
                    r0 = (NS - 1) * ZCH + q * CST
                    sz = CST if q < 3 else HALF - (NS - 1) * ZCH - 3 * CST
                    pltpu.sync_copy(acc.at[pl.ds(r0, sz)],
                                    stage.at[pl.ds(0, sz)])
                    pltpu.sync_copy(stage.at[pl.ds(0, sz)],
                                    out_hbm.at[pl.ds(base + r0, sz)])

    return agg


_agg1 = _make_agg(1)
_agg2 = _make_agg(2)


BM = 5000
NBLK = N // BM


def _tc1_body(x_ref, w_ref, cnt0_ref, cnt1_ref, u1a_ref, u1b_ref, dinv_ref):
    dv = lax.rsqrt(cnt0_ref[...] + cnt1_ref[...] + 1.0)
    u = jnp.dot(x_ref[...], w_ref[...], preferred_element_type=f32) * dv
    u1a_ref[...] = u[:, :32].astype(bf16)
    u1b_ref[...] = u[:, 32:].astype(bf16)
    dinv_ref[...] = dv


def _tc1(x, W1, cnt0, cnt1):
    return pl.pallas_call(
        _tc1_body,
        grid=(NBLK,),
        in_specs=[
            pl.BlockSpec((BM, 28), lambda i: (i, 0)),
            pl.BlockSpec((28, 64), lambda i: (0, 0)),
            pl.BlockSpec((BM, 1), lambda i: (i, 0)),
            pl.BlockSpec((BM, 1), lambda i: (i, 0)),
        ],
        out_specs=[
            pl.BlockSpec((BM, 32), lambda i: (i, 0)),
            pl.BlockSpec((BM, 32), lambda i: (i, 0)),
            pl.BlockSpec((BM, 1), lambda i: (i, 0)),
        ],
        out_shape=[
            jax.ShapeDtypeStruct((N, 32), bf16),
            jax.ShapeDtypeStruct((N, 32), bf16),
            jax.ShapeDtypeStruct((N, 1), f32),
        ],
    )(x, W1, cnt0, cnt1)


def _tc2_body(s1a, s1b, u1a, u1b, dinv, b1, w2, u2_ref):
    dv = dinv[...]
    ha = s1a[...].astype(f32) + u1a[...].astype(f32)
    hb = s1b[...].astype(f32) + u1b[...].astype(f32)
    h = jnp.tanh(jnp.concatenate([ha, hb], axis=1) * dv + b1[...])
    u2_ref[...] = (jnp.dot(h, w2[...], preferred_element_type=f32)
                   * dv).astype(bf16)


def _tc2(s1a, s1b, u1a, u1b, dinv, b1, W2):
    blk32 = pl.BlockSpec((BM, 32), lambda i: (i, 0))
    return pl.pallas_call(
        _tc2_body,
        grid=(NBLK,),
        in_specs=[
            blk32, blk32, blk32, blk32,
            pl.BlockSpec((BM, 1), lambda i: (i, 0)),
            pl.BlockSpec((1, 64), lambda i: (0, 0)),
            pl.BlockSpec((64, 32), lambda i: (0, 0)),
        ],
        out_specs=blk32,
        out_shape=jax.ShapeDtypeStruct((N, 32), bf16),
    )(s1a, s1b, u1a, u1b, dinv, b1, W2)


def _prod_rows(h):
    m = jnp.concatenate([h, jnp.ones((8192 - BM, 32), f32)], axis=0)
    n = 8192
    while n > 1:
        n //= 2
        m = m[:n] * m[n:2 * n]
    return m


def _tc3_body(s2, u2, dinv, b2, wd1, bd1, wd2, bd2, wo, bo,
              out_ref, g_ref, pacc):
    i = pl.program_id(0)
    h = jnp.tanh((s2[...].astype(f32) + u2[...].astype(f32)) * dinv[...]
                 + b2[...])
    part = _prod_rows(h)

    @pl.when(i == 0)
    def _():
        pacc[...] = part

    @pl.when(i > 0)
    def _():
        pacc[...] = pacc[...] * part

    @pl.when(i == NBLK - 1)
    def _():
        g1 = jnp.tanh(jnp.dot(pacc[...], wd1[...], preferred_element_type=f32)
                      + bd1[...])
        g2 = jnp.tanh(jnp.dot(g1, wd2[...], preferred_element_type=f32)
                      + bd2[...])
        out_ref[...] = jnp.dot(g2, wo[...], preferred_element_type=f32) + bo[...]
        g_ref[...] = g2


def _tc3(s2, u2, dinv, b2, Wd1, bd1, Wd2, bd2, Wo, bo):
    blk32 = pl.BlockSpec((BM, 32), lambda i: (i, 0))
    full = lambda a, b: pl.BlockSpec((a, b), lambda i: (0, 0))
    return pl.pallas_call(
        _tc3_body,
        grid=(NBLK,),
        in_specs=[
            blk32, blk32,
            pl.BlockSpec((BM, 1), lambda i: (i, 0)),
            full(1, 32), full(32, 128), full(1, 128),
            full(128, 64), full(1, 64), full(64, 1), full(1, 1),
        ],
        out_specs=[full(1, 1), full(1, 64)],
        out_shape=[
            jax.ShapeDtypeStruct((1, 1), f32),
            jax.ShapeDtypeStruct((1, 64), f32),
        ],
        scratch_shapes=[pltpu.VMEM((1, 32), f32)],
    )(s2, u2, dinv, b2, Wd1, bd1, Wd2, bd2, Wo, bo)


def kernel(x, edge_index, W1, b1, W2, b2, Wd1, bd1, Wd2, bd2, Wo, bo):
    row = edge_index[0]
    col = edge_index[1]
    cnt2 = _deg(col)
    u1a, u1b, dinv = _tc1(x, W1, cnt2[0].reshape(N, 1), cnt2[1].reshape(N, 1))
    s1a, s1b = _agg2(u1a, u1b, row, col)
    u2 = _tc2(s1a, s1b, u1a, u1b, dinv, b1.reshape(1, 64), W2)
    s2 = _agg1(u2, row, col)
    out, g = _tc3(s2, u2, dinv, b2.reshape(1, 32), Wd1, bd1.reshape(1, 128),
                  Wd2, bd2.reshape(1, 64), Wo, bo.reshape(1, 1))
    return (out, g)


# speedup vs baseline: 25.7867x; 1.4406x over previous
import functools

import jax
import jax.numpy as jnp
from jax import lax
from jax.experimental import pallas as pl
from jax.experimental.pallas import tpu as pltpu
from jax.experimental.pallas import tpu_sc as plsc

N = 100000
E = 1600000
HALF = 50000
TRASH = 50000
ACC_ROWS = 52096
CHUNK = 128
NCH = E // CHUNK
NS = 16
ZCH = 3256
NPAIR = 392
CST = 814
OCH = 3128

_mesh = plsc.VectorSubcoreMesh(core_axis_name="c", subcore_axis_name="s")
_sc_params = pltpu.CompilerParams(use_tc_tiling_on_sc=False)

f32 = jnp.float32
bf16 = jnp.bfloat16
i32 = jnp.int32


def _chunk_bounds(s):
    lo = (s * NCH + NS - 1) // NS
    hi = ((s + 1) * NCH + NS - 1) // NS
    return lo, hi


def _local_cols(cidx, h, cidx2, base, hib, trash=TRASH, tmask=2047):
    for j in range(CHUNK // 16):
        v = cidx[pl.ds(h * CHUNK + j * 16, 16)]
        inb = (v >= base) & (v < hib)
        cidx2[pl.ds(j * 16, 16)] = jnp.where(inb, v - base, trash + (v & tmask))


DNCH = NCH // 2
DACC = 100096
DZCH = 6256
DTRASH = N
DNPAIR = 196


@functools.partial(
    pl.kernel,
    out_type=jax.ShapeDtypeStruct((2, N), f32),
    mesh=_mesh,
    compiler_params=_sc_params,
    scratch_types=[
        pltpu.MemorySpace.VMEM_SHARED((DACC,), f32),
        pltpu.VMEM((2, 2 * CHUNK), i32),
        [pltpu.VMEM((CHUNK,), i32)] * 4,
        pltpu.VMEM((CHUNK,), f32),
        pltpu.VMEM((DZCH,), f32),
        [pltpu.SemaphoreType.DMA] * 2,
        [pltpu.SemaphoreType.DMA] * 2,
    ],
)
def _deg(col_hbm, cnt_hbm, acc, cidx, cidx2s, ones_v, stage, esems, ssems):
    c = lax.axis_index("c")
    s = lax.axis_index("s")

    def zstage(r, _):
        stage[pl.ds(r * 16, 16)] = jnp.zeros((16,), f32)
        return 0

    lax.fori_loop(0, DZCH // 16, zstage, 0)
    for j in range(CHUNK // 16):
        ones_v[pl.ds(j * 16, 16)] = jnp.ones((16,), f32)
    pltpu.sync_copy(stage, acc.at[pl.ds(s * DZCH, DZCH)])
    plsc.subcore_barrier()

    lo = c * DNCH + (s * DNCH + NS - 1) // NS
    hi = c * DNCH + ((s + 1) * DNCH + NS - 1) // NS
    size = hi - lo

    def eload(p, b):
        cp = jnp.minimum(lo + 2 * p, (c + 1) * DNCH - 2) * CHUNK
        pltpu.async_copy(col_hbm.at[pl.ds(cp, 2 * CHUNK)], cidx.at[b],
                         esems[b])

    def ewait(b):
        pltpu.make_async_copy(col_hbm.at[pl.ds(0, 2 * CHUNK)], cidx.at[b],
                              esems[b]).wait()

    def compute(p, b):
        for h in range(2):
            hib = jnp.where(2 * p + h < size, N, 0)
            _local_cols(cidx.at[b], h, cidx2s[2 * b + h], 0, hib, DTRASH, 63)

    def fire_scatters(b):
        for h in range(2):
            pltpu.async_copy(ones_v, acc.at[cidx2s[2 * b + h]], ssems[b],
                             add=True)

    def swait(b):
        for h in range(2):
            pltpu.make_async_copy(ones_v, acc.at[cidx2s[2 * b + h]],
                                  ssems[b]).wait()

    eload(0, 0)
    eload(1, 1)
    ewait(0)
    compute(0, 0)
    fire_scatters(0)
    ewait(1)
    compute(1, 1)
    fire_scatters(1)
    eload(2, 0)

    def body(g, _):
        for b in range(2):
            p = 2 * g + 2 + b
            ewait(b)
            swait(b)
            compute(p, b)
            fire_scatters(b)
            eload(p + 1, 1 - b)
        return 0

    lax.fori_loop(0, (DNPAIR - 2) // 2, body, 0)
    swait(0)
    swait(1)
    ewait(0)
    plsc.subcore_barrier()

    @pl.when(s < NS - 1)
    def _():
        pltpu.sync_copy(acc.at[pl.ds(s * DZCH, DZCH)], stage)
        pltpu.sync_copy(stage, cnt_hbm.at[c, pl.ds(s * DZCH, DZCH)])

    @pl.when(s == NS - 1)
    def _():
        rem = N - (NS - 1) * DZCH
        pltpu.sync_copy(acc.at[pl.ds((NS - 1) * DZCH, rem)], stage.at[pl.ds(0, rem)])
        pltpu.sync_copy(stage.at[pl.ds(0, rem)],
                        cnt_hbm.at[c, pl.ds((NS - 1) * DZCH, rem)])


def _make_agg(npass):
    out_t = [jax.ShapeDtypeStruct((N, 32), bf16)] * npass

    @functools.partial(
        pl.kernel,
        out_type=tuple(out_t) if npass > 1 else out_t[0],
        mesh=_mesh,
        compiler_params=_sc_params,
        scratch_types=[
            pltpu.MemorySpace.VMEM_SHARED((ACC_ROWS, 32), bf16),
            pltpu.VMEM((2, 2 * CHUNK), i32),
            pltpu.VMEM((2, 2 * CHUNK), i32),
            [pltpu.VMEM((CHUNK,), i32)] * 4,
            [pltpu.VMEM((CHUNK, 32), bf16)] * 4,
            pltpu.VMEM((CST, 32), bf16),
            [pltpu.SemaphoreType.DMA] * 2,
            [pltpu.SemaphoreType.DMA] * 2,
            [pltpu.SemaphoreType.DMA] * 2,
        ],
    )
    def agg(*refs):
        tables = refs[:npass]
        row_hbm, col_hbm = refs[npass], refs[npass + 1]
        outs = refs[npass + 2:npass + 2 + npass]
        (acc, ridx, cidx, cidx2s, rowss, stage, esems, gsems,
         ssems) = refs[npass + 2 + npass:]
        c = lax.axis_index("c")
        s = lax.axis_index("s")
        base = c * HALF
        lo, hi = _chunk_bounds(s)
        size = hi - lo

        def eload(p, b):
            cp = jnp.minimum(lo + 2 * p, NCH - 2) * CHUNK
            pltpu.async_copy(row_hbm.at[pl.ds(cp, 2 * CHUNK)], ridx.at[b],
                             esems[b])
            pltpu.async_copy(col_hbm.at[pl.ds(cp, 2 * CHUNK)], cidx.at[b],
                             esems[b])

        def ewait(b):
            pltpu.make_async_copy(row_hbm.at[pl.ds(0, 2 * CHUNK)], ridx.at[b],
                                  esems[b]).wait()
            pltpu.make_async_copy(col_hbm.at[pl.ds(0, 2 * CHUNK)], cidx.at[b],
                                  esems[b]).wait()

        def zstage(r, _):
            stage[r, pl.ds(0, 32)] = jnp.zeros((32,), bf16)
            return 0

        for p in range(npass):
            table, out_hbm = tables[p], outs[p]
            if p:
                plsc.subcore_barrier()
            lax.fori_loop(0, CST, zstage, 0)
            for q in range(4):
                pltpu.sync_copy(stage, acc.at[pl.ds(s * ZCH + q * CST, CST)])
            plsc.subcore_barrier()

            def compute_and_gather(pp, b):
                ewait(b)
                for h in range(2):
                    hib = jnp.where(2 * pp + h < size, base + HALF, base)
                    _local_cols(cidx.at[b], h, cidx2s[2 * b + h], base, hib)
                for h in range(2):
                    pltpu.async_copy(
                        table.at[ridx.at[b, pl.ds(h * CHUNK, CHUNK)]],
                        rowss[2 * b + h], gsems[b])

            def gwait(b):
                for h in range(2):
                    pltpu.make_async_copy(
                        table.at[ridx.at[b, pl.ds(0, CHUNK)]],
                        rowss[2 * b + h], gsems[b]).wait()

            def fire_scatters(b):
                for h in range(2):
                    pltpu.async_copy(rowss[2 * b + h],
                                     acc.at[cidx2s[2 * b + h]], ssems[b],
                                     add=True)

            def swait(b):
                for h in range(2):
                    pltpu.make_async_copy(rowss[2 * b + h],
                                          acc.at[cidx2s[2 * b + h]],
                                          ssems[b]).wait()

            eload(0, 0)
            eload(1, 1)
            compute_and_gather(0, 0)
            compute_and_gather(1, 1)
            gwait(0)
            fire_scatters(0)
            eload(2, 0)

            def body(g, _):
                for b in range(2):
                    pp = 2 * g + 2 + b
                    swait(b)
                    compute_and_gather(pp, b)
                    gwait(1 - b)
                    fire_scatters(1 - b)
                    eload(pp + 1, 1 - b)
                return 0

            lax.fori_loop(0, (NPAIR - 2) // 2, body, 0)

            gwait(1)
            fire_scatters(1)
            swait(0)
            swait(1)
            ewait(0)
            plsc.subcore_barrier()

            @pl.when(s < NS - 1)
            def _():
                for q in range(4):
                    r0 = s * OCH + q * 782
                    pltpu.sync_copy(acc.at[pl.ds(r0, 782)],
                                    stage.at[pl.ds(0, 782)])
                    pltpu.sync_copy(stage.at[pl.ds(0, 782)],
                                    out_hbm.at[pl.ds(base + r0, 782)])

            @pl.when(s == NS - 1)
            def _():
                for q in range(4):
                    r0 = (NS - 1) * OCH + q * 782
                    sz = 782 if q < 3 else HALF - (NS - 1) * OCH - 3 * 782
                    pltpu.sync_copy(acc.at[pl.ds(r0, sz)],
                                    stage.at[pl.ds(0, sz)])
                    pltpu.sync_copy(stage.at[pl.ds(0, sz)],
                                    out_hbm.at[pl.ds(base + r0, sz)])

    return agg


_agg1 = _make_agg(1)
_agg2 = _make_agg(2)


BM = 5000
NBLK = N // BM


def _tc1_body(x_ref, w_ref, cnt0_ref, cnt1_ref, u1a_ref, u1b_ref, dinv_ref):
    dv = lax.rsqrt(cnt0_ref[...] + cnt1_ref[...] + 1.0)
    u = jnp.dot(x_ref[...], w_ref[...], preferred_element_type=f32) * dv
    u1a_ref[...] = u[:, :32].astype(bf16)
    u1b_ref[...] = u[:, 32:].astype(bf16)
    dinv_ref[...] = dv


def _tc1(x, W1, cnt0, cnt1):
    return pl.pallas_call(
        _tc1_body,
        grid=(NBLK,),
        in_specs=[
            pl.BlockSpec((BM, 28), lambda i: (i, 0)),
            pl.BlockSpec((28, 64), lambda i: (0, 0)),
            pl.BlockSpec((BM, 1), lambda i: (i, 0)),
            pl.BlockSpec((BM, 1), lambda i: (i, 0)),
        ],
        out_specs=[
            pl.BlockSpec((BM, 32), lambda i: (i, 0)),
            pl.BlockSpec((BM, 32), lambda i: (i, 0)),
            pl.BlockSpec((BM, 1), lambda i: (i, 0)),
        ],
        out_shape=[
            jax.ShapeDtypeStruct((N, 32), bf16),
            jax.ShapeDtypeStruct((N, 32), bf16),
            jax.ShapeDtypeStruct((N, 1), f32),
        ],
    )(x, W1, cnt0, cnt1)


def _tc2_body(s1a, s1b, u1a, u1b, dinv, b1, w2, u2_ref):
    dv = dinv[...]
    ha = s1a[...].astype(f32) + u1a[...].astype(f32)
    hb = s1b[...].astype(f32) + u1b[...].astype(f32)
    h = jnp.tanh(jnp.concatenate([ha, hb], axis=1) * dv + b1[...])
    u2_ref[...] = (jnp.dot(h, w2[...], preferred_element_type=f32)
                   * dv).astype(bf16)


def _tc2(s1a, s1b, u1a, u1b, dinv, b1, W2):
    blk32 = pl.BlockSpec((BM, 32), lambda i: (i, 0))
    return pl.pallas_call(
        _tc2_body,
        grid=(NBLK,),
        in_specs=[
            blk32, blk32, blk32, blk32,
            pl.BlockSpec((BM, 1), lambda i: (i, 0)),
            pl.BlockSpec((1, 64), lambda i: (0, 0)),
            pl.BlockSpec((64, 32), lambda i: (0, 0)),
        ],
        out_specs=blk32,
        out_shape=jax.ShapeDtypeStruct((N, 32), bf16),
    )(s1a, s1b, u1a, u1b, dinv, b1, W2)


def _prod_rows(h):
    m = jnp.concatenate([h, jnp.ones((8192 - BM, 32), f32)], axis=0)
    n = 8192
    while n > 1:
        n //= 2
        m = m[:n] * m[n:2 * n]
    return m


def _tc3_body(s2, u2, dinv, b2, wd1, bd1, wd2, bd2, wo, bo,
              out_ref, g_ref, pacc):
    i = pl.program_id(0)
    h = jnp.tanh((s2[...].astype(f32) + u2[...].astype(f32)) * dinv[...]
                 + b2[...])
    part = _prod_rows(h)

    @pl.when(i == 0)
    def _():
        pacc[...] = part

    @pl.when(i > 0)
    def _():
        pacc[...] = pacc[...] * part

    @pl.when(i == NBLK - 1)
    def _():
        g1 = jnp.tanh(jnp.dot(pacc[...], wd1[...], preferred_element_type=f32)
                      + bd1[...])
        g2 = jnp.tanh(jnp.dot(g1, wd2[...], preferred_element_type=f32)
                      + bd2[...])
        out_ref[...] = jnp.dot(g2, wo[...], preferred_element_type=f32) + bo[...]
        g_ref[...] = g2


def _tc3(s2, u2, dinv, b2, Wd1, bd1, Wd2, bd2, Wo, bo):
    blk32 = pl.BlockSpec((BM, 32), lambda i: (i, 0))
    full = lambda a, b: pl.BlockSpec((a, b), lambda i: (0, 0))
    return pl.pallas_call(
        _tc3_body,
        grid=(NBLK,),
        in_specs=[
            blk32, blk32,
            pl.BlockSpec((BM, 1), lambda i: (i, 0)),
            full(1, 32), full(32, 128), full(1, 128),
            full(128, 64), full(1, 64), full(64, 1), full(1, 1),
        ],
        out_specs=[full(1, 1), full(1, 64)],
        out_shape=[
            jax.ShapeDtypeStruct((1, 1), f32),
            jax.ShapeDtypeStruct((1, 64), f32),
        ],
        scratch_shapes=[pltpu.VMEM((1, 32), f32)],
    )(s2, u2, dinv, b2, Wd1, bd1, Wd2, bd2, Wo, bo)


def kernel(x, edge_index, W1, b1, W2, b2, Wd1, bd1, Wd2, bd2, Wo, bo):
    row = edge_index[0]
    col = edge_index[1]
    cnt2 = _deg(col)
    u1a, u1b, dinv = _tc1(x, W1, cnt2[0].reshape(N, 1), cnt2[1].reshape(N, 1))
    s1a, s1b = _agg2(u1a, u1b, row, col)
    u2 = _tc2(s1a, s1b, u1a, u1b, dinv, b1.reshape(1, 64), W2)
    s2 = _agg1(u2, row, col)
    out, g = _tc3(s2, u2, dinv, b2.reshape(1, 32), Wd1, bd1.reshape(1, 128),
                  Wd2, bd2.reshape(1, 64), Wo, bo.reshape(1, 1))
    return (out, g)
